# baseline (speedup 1.0000x reference)
import jax
import jax.numpy as jnp
from jax import lax
from jax.experimental import pallas as pl
from jax.experimental.pallas import tpu as pltpu
from jax.experimental.pallas import tpu_sc as plsc

_N = 10000
_NPAD = 10112
_NC, _NS = 2, 16
_NW = _NC * _NS
_CH = 128
_NBUF = 4
_KWIN = 20
_KFULL = 80
_RPT = _NPAD // _NS
_D = 32

_mesh = plsc.VectorSubcoreMesh(core_axis_name="c", subcore_axis_name="s")
_sc_params = pltpu.CompilerParams(use_tc_tiling_on_sc=False)


def _deg_body(d0, d1, d2, d3, ones_hbm, zeros_hbm, out, idx_v, ones_v, acc):
    c = lax.axis_index("c")
    s = lax.axis_index("s")
    wid = c * _NS + s
    r0 = s * _RPT
    pltpu.sync_copy(ones_hbm, ones_v)
    for w in range(4):
        pltpu.sync_copy(zeros_hbm, acc.at[w, pl.ds(r0, _RPT)])
    plsc.subcore_barrier()
    for w, dref in enumerate((d0, d1, d2, d3)):
        pltpu.sync_copy(dref.at[wid], idx_v)

        def _one(j, carry, _w=w):
            pltpu.sync_copy(ones_v, acc.at[_w].at[idx_v.at[j]], add=True)
            return carry

        lax.fori_loop(0, _KWIN, _one, 0)
    plsc.subcore_barrier()
    for w in range(4):
        pltpu.sync_copy(acc.at[w, pl.ds(r0, _RPT)],
                        out.at[c, w, pl.ds(r0, _RPT)])


_deg_call = pl.kernel(
    _deg_body,
    out_type=jax.ShapeDtypeStruct((_NC, 4, _NPAD), jnp.float32),
    mesh=_mesh,
    compiler_params=_sc_params,
    scratch_types=[
        pltpu.VMEM((_KWIN, _CH), jnp.int32),
        pltpu.VMEM((_CH,), jnp.float32),
        pltpu.VMEM_SHARED((4, _NPAD), jnp.float32),
    ],
)


def _make_conv_call(K):
    def body(src, dst, hp, zeros_hbm, out, isv, idv, *rest):
        rows = rest[:_NBUF]
        gsem = rest[_NBUF: 2 * _NBUF]
        ssem = rest[2 * _NBUF: 3 * _NBUF]
        acc = rest[3 * _NBUF]

        c = lax.axis_index("c")
        s = lax.axis_index("s")
        wid = c * _NS + s
        r0 = s * _RPT

        @pl.when(c == 0)
        def _():
            pltpu.sync_copy(hp.at[pl.ds(r0, _RPT)], acc.at[pl.ds(r0, _RPT)])

        @pl.when(c != 0)
        def _():
            pltpu.sync_copy(zeros_hbm, acc.at[pl.ds(r0, _RPT)])

        plsc.subcore_barrier()

        pltpu.sync_copy(src.at[wid], isv)
        pltpu.sync_copy(dst.at[wid], idv)
        for b in range(_NBUF):
            pltpu.async_copy(hp.at[isv.at[b]], rows[b], gsem[b])

        def _group(g, carry):
            j0 = _NBUF * g
            for b in range(_NBUF):
                pltpu.make_async_copy(hp.at[isv.at[j0 + b]],
                                      rows[b], gsem[b]).wait()
                pltpu.async_copy(rows[b], acc.at[idv.at[j0 + b]],
                                 ssem[b], add=True)
            for b in range(_NBUF):
                @pl.when(j0 + b + _NBUF < K)
                def _(b=b):
                    pltpu.make_async_copy(rows[b], acc.at[idv.at[j0 + b]],
                                          ssem[b]).wait()
                    pltpu.async_copy(hp.at[isv.at[j0 + b + _NBUF]],
                                     rows[b], gsem[b])
            return carry

        lax.fori_loop(0, K // _NBUF, _group, 0)
        for b in range(_NBUF):
            pltpu.make_async_copy(rows[b], acc.at[idv.at[K - _NBUF + b]],
                                  ssem[b]).wait()

        plsc.subcore_barrier()
        pltpu.sync_copy(acc.at[pl.ds(r0, _RPT)],
                        out.at[c, pl.ds(r0, _RPT)])

    scratch = [
        pltpu.VMEM((K, _CH), jnp.int32),
        pltpu.VMEM((K, _CH), jnp.int32),
    ]
    scratch += [pltpu.VMEM((_CH, _D), jnp.float32) for _ in range(_NBUF)]
    scratch += [pltpu.SemaphoreType.DMA for _ in range(2 * _NBUF)]
    scratch += [pltpu.VMEM_SHARED((_NPAD, _D), jnp.float32)]

    return pl.kernel(
        body,
        out_type=jax.ShapeDtypeStruct((_NC, _NPAD, _D), jnp.float32),
        mesh=_mesh,
        compiler_params=_sc_params,
        scratch_types=scratch,
    )


_conv_win = _make_conv_call(_KWIN)
_conv_full = _make_conv_call(_KFULL)


def _epi(acc_ref, dis_col, bias, g, bvec):
    y = dis_col * (acc_ref[0] + acc_ref[1]) + bias[None, :]
    yv = y[:_N]
    m = jnp.mean(yv, axis=0)
    var = jnp.mean(yv * yv, axis=0) - m * m
    scale = lax.rsqrt(var + 1e-5) * g
    return (y - m[None, :]) * scale[None, :] + bvec[None, :]


_hp_t = jax.ShapeDtypeStruct((_NPAD, _D), jnp.float32)


def _tc0a_body(xp, l1w, l1b, cw, cfw, cbw, h_a, h_f1, h_b4):
    x1 = xp[...] @ l1w[...] + l1b[...][None, :]
    h_a[...] = x1 @ cw[...]
    h_f1[...] = x1 @ cfw[...]
    h_b4[...] = x1 @ cbw[...]


_tc0a = pl.pallas_call(_tc0a_body, out_shape=(_hp_t, _hp_t, _hp_t))


def _tc0b_body(degp, h_a, h_f1, h_b4, hp_a, hp_f1, hp_b4, dis8):
    dsum = degp[0] + degp[1]
    degf = dsum[0:1] + dsum[1:2] + dsum[2:3] + dsum[3:4] - 3.0
    dis = lax.rsqrt(jnp.concatenate(
        [dsum, degf, jnp.ones((3, _NPAD), jnp.float32)], axis=0))
    d8 = dis.T
    dis8[...] = d8
    hp_a[...] = d8[:, 4:5] * h_a[...]
    hp_f1[...] = d8[:, 0:1] * h_f1[...]
    hp_b4[...] = d8[:, 3:4] * h_b4[...]


_tc0b = pl.pallas_call(
    _tc0b_body,
    out_shape=(_hp_t, _hp_t, _hp_t,
               jax.ShapeDtypeStruct((_NPAD, 8), jnp.float32)),
)


def _make_tc_step(ecol, ncol):
    def body(acc, dis8, cb, g, bv, w, hp_n):
        d8 = dis8[...]
        xn = _epi(acc, d8[:, ecol:ecol + 1], cb[...], g[...], bv[...])
        hp_n[...] = d8[:, ncol:ncol + 1] * (xn @ w[...])

    return pl.pallas_call(body, out_shape=_hp_t)


_tc_f1 = _make_tc_step(0, 1)
_tc_f2 = _make_tc_step(1, 2)
_tc_f3 = _make_tc_step(2, 3)
_tc_b4 = _make_tc_step(3, 2)
_tc_b3 = _make_tc_step(2, 1)
_tc_b2 = _make_tc_step(1, 0)


def _make_tc_epi(ecol):
    def body(acc, dis8, cb, g, bv, xn_o):
        xn_o[...] = _epi(acc, dis8[...][:, ecol:ecol + 1], cb[...], g[...],
                         bv[...])

    return pl.pallas_call(body, out_shape=_hp_t)


_tc_a = _make_tc_epi(4)
_tc_f4 = _make_tc_epi(3)


def _tc_fin_body(acc, dis8, cb, g, bv, xa, xf, l2w, l2b, out):
    xb = _epi(acc, dis8[...][:, 0:1], cb[...], g[...], bv[...])
    cat = jnp.concatenate([xa[...][:_N], xf[...][:_N], xb[:_N]], axis=1)
    out[...] = cat @ l2w[...] + l2b[...][None, :]


_tc_fin = pl.pallas_call(
    _tc_fin_body, out_shape=jax.ShapeDtypeStruct((_N, _D), jnp.float32))


def _pad_split(a, K):
    tot = _NW * K * _CH
    pad = jnp.full((tot - a.shape[0],), _N, jnp.int32)
    return jnp.concatenate([a.astype(jnp.int32), pad]).reshape(_NW, K, _CH)


def kernel(x, edge_index, lin1_w, lin1_b, conv_w, conv_b, convf_w, convf_b,
           convb_w, convb_b, bn_g, bn_b, bnf_g, bnf_b, bnb_g, bnb_b,
           lin2_w, lin2_b):
    ei = edge_index.astype(jnp.int32)
    src_w = [_pad_split(ei[0, w * 80000:(w + 1) * 80000], _KWIN) for w in range(4)]
    dst_w = [_pad_split(ei[1, w * 80000:(w + 1) * 80000], _KWIN) for w in range(4)]
    src_f = _pad_split(ei[0], _KFULL)
    dst_f = _pad_split(ei[1], _KFULL)

    zeros32 = jnp.zeros((_RPT, _D), jnp.float32)
    zeros1 = jnp.zeros((_RPT,), jnp.float32)
    ones1 = jnp.ones((_CH,), jnp.float32)
    xp = jnp.concatenate([x, jnp.zeros((_NPAD - _N, x.shape[1]), x.dtype)], axis=0)

    degp = _deg_call(dst_w[0], dst_w[1], dst_w[2], dst_w[3], ones1, zeros1)
    h_a, h_f1, h_b4 = _tc0a(xp, lin1_w, lin1_b, conv_w, convf_w, convb_w)
    hp_a, hp_f1, hp_b4, dis8 = _tc0b(degp, h_a, h_f1, h_b4)

    acc_f1 = _conv_win(src_w[0], dst_w[0], hp_f1, zeros32)
    acc_b4 = _conv_win(src_w[3], dst_w[3], hp_b4, zeros32)
    acc_a = _conv_full(src_f, dst_f, hp_a, zeros32)

    hp_f2 = _tc_f1(acc_f1, dis8, convf_b, bnf_g, bnf_b, convf_w)
    hp_b3 = _tc_b4(acc_b4, dis8, convb_b, bnb_g, bnb_b, convb_w)
    xa = _tc_a(acc_a, dis8, conv_b, bn_g, bn_b)

    acc_f2 = _conv_win(src_w[1], dst_w[1], hp_f2, zeros32)
    acc_b3 = _conv_win(src_w[2], dst_w[2], hp_b3, zeros32)
    hp_f3 = _tc_f2(acc_f2, dis8, convf_b, bnf_g, bnf_b, convf_w)
    hp_b2 = _tc_b3(acc_b3, dis8, convb_b, bnb_g, bnb_b, convb_w)

    acc_f3 = _conv_win(src_w[2], dst_w[2], hp_f3, zeros32)
    acc_b2 = _conv_win(src_w[1], dst_w[1], hp_b2, zeros32)
    hp_f4 = _tc_f3(acc_f3, dis8, convf_b, bnf_g, bnf_b, convf_w)
    hp_b1 = _tc_b2(acc_b2, dis8, convb_b, bnb_g, bnb_b, convb_w)

    acc_f4 = _conv_win(src_w[3], dst_w[3], hp_f4, zeros32)
    acc_b1 = _conv_win(src_w[0], dst_w[0], hp_b1, zeros32)
    xf = _tc_f4(acc_f4, dis8, convf_b, bnf_g, bnf_b)

    return _tc_fin(acc_b1, dis8, convb_b, bnb_g, bnb_b,
                   xa, xf, lin2_w, lin2_b)
